# Initial kernel scaffold; baseline (speedup 1.0000x reference)
import jax
import jax.numpy as jnp
from jax.experimental import pallas as pl


def kernel(tracks, event_id_map, unique_ids, unique_eventIDs, fields):
    raise NotImplementedError("write your pallas kernel here")


# speedup vs baseline: 71.6464x; 71.6464x over previous
import functools

import jax
import jax.numpy as jnp
from jax import lax
from jax.experimental import pallas as pl
from jax.experimental.pallas import tpu as pltpu
from jax.experimental.pallas import tpu_sc as plsc

GRID = 256
NPIX = GRID * GRID
T_SIG = 50
T_TOTAL = 500
N_ADC = 10
ROW_W = 16
NCORES = 2
NSUB = 16
NW = NCORES * NSUB
BATCH = 128
ZROWS = 1024


def _track_stage(data_ref, vals_ref, rows_ref):
    d = data_ref[...]
    x = d[:, 0:1]
    y = d[:, 1:2]
    t_drift = jnp.abs(d[:, 2:3])
    dE = jnp.abs(d[:, 3:4])
    dEdx = jnp.abs(d[:, 4:5]) + 0.1
    ev = d[:, 5:6]

    recomb = 0.8 / (1.0 + 0.0486 * dEdx)
    dQ = recomb * dE / 2.36e-05 * 0.0001
    dQ = dQ * jnp.exp(-t_drift / 2.2)

    px = jnp.clip(((x + 5.0) / 10.0 * GRID).astype(jnp.int32), 2, GRID - 3)
    py = jnp.clip(((y + 5.0) / 10.0 * GRID).astype(jnp.int32), 2, GRID - 3)
    pxf = px.astype(jnp.float32)
    pyf = py.astype(jnp.float32)

    B = d.shape[0]
    p_id = lax.broadcasted_iota(jnp.int32, (1, 16), 1).astype(jnp.float32)
    offa = jnp.floor(p_id / 4.0) - 2.0
    offb = p_id - 4.0 * jnp.floor(p_id / 4.0) - 2.0
    nxf = pxf + offa
    nyf = pyf + offb

    cx = (nxf + 0.5) / GRID * 10.0 - 5.0
    cy = (nyf + 0.5) / GRID * 10.0 - 5.0
    d2 = (cx - x) ** 2 + (cy - y) ** 2
    w = jnp.exp(-d2 / 0.08)
    w = w / (jnp.sum(w, axis=1, keepdims=True) + 1e-09)

    rows_ref[...] = (nxf * GRID + nyf).astype(jnp.int32)

    tk = lax.broadcasted_iota(jnp.int32, (B, T_SIG), 1).astype(jnp.float32)
    t_pk = jnp.clip(t_drift * 5.0, 5.0, 45.0)
    sig = 1.0 + 0.5 * t_drift
    g = jnp.exp(-((tk - t_pk) ** 2) / (2.0 * sig * sig))
    gden = jnp.sum(g, axis=1, keepdims=True) + 1e-09
    gn = g / gden
    step = (T_TOTAL - T_SIG) // 100
    stride = T_TOTAL // N_ADC
    pcols = []
    for k in range(N_ADC):
        ck = (stride - 1 + stride * k) - step * ev
        pcols.append(jnp.sum(jnp.where(tk <= ck, gn, 0.0), axis=1,
                             keepdims=True))
    for _ in range(ROW_W - N_ADC):
        pcols.append(jnp.zeros((B, 1), jnp.float32))
    pref = jnp.concatenate(pcols, axis=1)

    for p in range(16):
        amp = dQ * w[:, p:p + 1]
        vals_ref[:, p * ROW_W:(p + 1) * ROW_W] = amp * pref


def _make_scatter(nchunks_per_worker):
    mesh = plsc.VectorSubcoreMesh(core_axis_name="c", subcore_axis_name="s")

    @functools.partial(
        pl.kernel,
        mesh=mesh,
        out_type=jax.ShapeDtypeStruct((NCORES, NPIX, ROW_W), jnp.float32),
        scratch_types=[
            pltpu.VMEM((BATCH, ROW_W), jnp.float32),
            pltpu.VMEM((BATCH,), jnp.int32),
            pltpu.VMEM((ZROWS, ROW_W), jnp.float32),
            pltpu.VMEM_SHARED((NPIX, ROW_W), jnp.float32),
        ],
        compiler_params=pltpu.CompilerParams(use_tc_tiling_on_sc=False),
    )
    def scatter_kernel(vals_hbm, rows_hbm, out_hbm, vbuf, ibuf, zbuf, table):
        c = lax.axis_index("c")
        s = lax.axis_index("s")
        wid = c * NSUB + s
        sl = NPIX // NSUB

        def zero_row(i, _):
            zbuf[i, :] = jnp.zeros((ROW_W,), jnp.float32)
            return 0

        lax.fori_loop(0, ZROWS, zero_row, 0)
        for j in range(sl // ZROWS):
            pltpu.sync_copy(zbuf, table.at[pl.ds(s * sl + j * ZROWS, ZROWS)])
        plsc.subcore_barrier()

        base = wid * nchunks_per_worker
        for b in range(nchunks_per_worker):
            pltpu.sync_copy(vals_hbm.at[base + b], vbuf)
            pltpu.sync_copy(rows_hbm.at[base + b], ibuf)
            pltpu.sync_copy(vbuf, table.at[ibuf], add=True)
        plsc.subcore_barrier()

        pltpu.sync_copy(table.at[pl.ds(s * sl, sl)],
                        out_hbm.at[c, pl.ds(s * sl, sl)])

    return scatter_kernel


def _make_gather(upad, nchunks_per_worker):
    mesh = plsc.VectorSubcoreMesh(core_axis_name="c", subcore_axis_name="s")

    @functools.partial(
        pl.kernel,
        mesh=mesh,
        out_type=jax.ShapeDtypeStruct((upad, ROW_W), jnp.float32),
        scratch_types=[
            pltpu.VMEM((BATCH,), jnp.int32),
            pltpu.VMEM((BATCH, ROW_W), jnp.float32),
            pltpu.VMEM((BATCH, ROW_W), jnp.float32),
            pltpu.VMEM((BATCH, ROW_W), jnp.float32),
            pltpu.SemaphoreType.DMA,
        ],
        compiler_params=pltpu.CompilerParams(use_tc_tiling_on_sc=False),
    )
    def gather_kernel(ta_hbm, tb_hbm, uid_hbm, out_hbm,
                      ibuf, abuf, bbuf, obuf, sem):
        c = lax.axis_index("c")
        s = lax.axis_index("s")
        wid = c * NSUB + s
        for j in range(nchunks_per_worker):
            row0 = (j * NW + wid) * BATCH
            pltpu.sync_copy(uid_hbm.at[pl.ds(row0, BATCH)], ibuf)
            pltpu.async_copy(ta_hbm.at[ibuf], abuf, sem).wait()
            pltpu.async_copy(tb_hbm.at[ibuf], bbuf, sem).wait()

            def finalize_row(r, _):
                v = (abuf[r, :] + bbuf[r, :]) * 4.0 + 74.0
                obuf[r, :] = jnp.clip(v, 0.0, 255.0)
                return 0

            lax.fori_loop(0, BATCH, finalize_row, 0)
            pltpu.sync_copy(obuf, out_hbm.at[pl.ds(row0, BATCH)])

    return gather_kernel


def kernel(tracks, event_id_map, unique_ids, unique_eventIDs, fields):
    n = tracks.shape[0]
    npad = ((n + 2047) // 2048) * 2048
    u = unique_ids.shape[0]
    upad = ((u + NW * BATCH - 1) // (NW * BATCH)) * (NW * BATCH)

    data = jnp.zeros((npad, 16), jnp.float32)
    data = data.at[:n, :5].set(jnp.take(tracks, fields[:5], axis=1))
    data = data.at[:n, 5].set(event_id_map.astype(jnp.float32))

    bt = 2048
    vals, rows = pl.pallas_call(
        _track_stage,
        grid=(npad // bt,),
        in_specs=[pl.BlockSpec((bt, 16), lambda i: (i, 0))],
        out_specs=[
            pl.BlockSpec((bt, 16 * ROW_W), lambda i: (i, 0)),
            pl.BlockSpec((bt, 16), lambda i: (i, 0)),
        ],
        out_shape=[
            jax.ShapeDtypeStruct((npad, 16 * ROW_W), jnp.float32),
            jax.ShapeDtypeStruct((npad, 16), jnp.int32),
        ],
    )(data)

    ncontrib = npad * 16
    nchunks = ncontrib // BATCH
    vals_r = vals.reshape(nchunks, BATCH, ROW_W)
    rows_r = rows.reshape(nchunks, BATCH)

    tables = _make_scatter(nchunks // NW)(vals_r, rows_r)

    uid_pad = jnp.zeros((upad,), jnp.int32).at[:u].set(unique_ids)
    out16 = _make_gather(upad, upad // (NW * BATCH))(
        tables[0], tables[1], uid_pad)
    return out16[:u, :N_ADC]


# speedup vs baseline: 119.9579x; 1.6743x over previous
import functools

import jax
import jax.numpy as jnp
from jax import lax
from jax.experimental import pallas as pl
from jax.experimental.pallas import tpu as pltpu
from jax.experimental.pallas import tpu_sc as plsc

GRID = 256
NPIX = GRID * GRID
T_SIG = 50
T_TOTAL = 500
N_ADC = 10
ROW_W = 16
NCORES = 2
NSUB = 16
NW = NCORES * NSUB
BATCH = 128
ZROWS = 1024


def _track_stage(data_ref, vals_ref, rows_ref):
    d = data_ref[...]
    x = d[:, 0:1]
    y = d[:, 1:2]
    t_drift = jnp.abs(d[:, 2:3])
    dE = jnp.abs(d[:, 3:4])
    dEdx = jnp.abs(d[:, 4:5]) + 0.1
    ev = d[:, 5:6]

    recomb = 0.8 / (1.0 + 0.0486 * dEdx)
    dQ = recomb * dE / 2.36e-05 * 0.0001
    dQ = dQ * jnp.exp(-t_drift / 2.2)

    px = jnp.clip(((x + 5.0) / 10.0 * GRID).astype(jnp.int32), 2, GRID - 3)
    py = jnp.clip(((y + 5.0) / 10.0 * GRID).astype(jnp.int32), 2, GRID - 3)
    pxf = px.astype(jnp.float32)
    pyf = py.astype(jnp.float32)

    B = d.shape[0]
    p_id = lax.broadcasted_iota(jnp.int32, (1, 16), 1).astype(jnp.float32)
    offa = jnp.floor(p_id / 4.0) - 2.0
    offb = p_id - 4.0 * jnp.floor(p_id / 4.0) - 2.0
    nxf = pxf + offa
    nyf = pyf + offb

    cx = (nxf + 0.5) / GRID * 10.0 - 5.0
    cy = (nyf + 0.5) / GRID * 10.0 - 5.0
    d2 = (cx - x) ** 2 + (cy - y) ** 2
    w = jnp.exp(-d2 / 0.08)
    w = w / (jnp.sum(w, axis=1, keepdims=True) + 1e-09)

    rows_ref[...] = (nxf * GRID + nyf).astype(jnp.int32)

    tk = lax.broadcasted_iota(jnp.int32, (B, T_SIG), 1).astype(jnp.float32)
    t_pk = jnp.clip(t_drift * 5.0, 5.0, 45.0)
    sig = 1.0 + 0.5 * t_drift
    g = jnp.exp(-((tk - t_pk) ** 2) / (2.0 * sig * sig))
    gden = jnp.sum(g, axis=1, keepdims=True) + 1e-09
    gn = g / gden
    step = (T_TOTAL - T_SIG) // 100
    stride = T_TOTAL // N_ADC
    pcols = []
    for k in range(N_ADC):
        ck = (stride - 1 + stride * k) - step * ev
        pcols.append(jnp.sum(jnp.where(tk <= ck, gn, 0.0), axis=1,
                             keepdims=True))
    for _ in range(ROW_W - N_ADC):
        pcols.append(jnp.zeros((B, 1), jnp.float32))
    pref = jnp.concatenate(pcols, axis=1)

    for p in range(16):
        amp = dQ * w[:, p:p + 1]
        vals_ref[:, p * ROW_W:(p + 1) * ROW_W] = amp * pref


def _make_scatter(nchunks_per_worker):
    mesh = plsc.VectorSubcoreMesh(core_axis_name="c", subcore_axis_name="s")

    @functools.partial(
        pl.kernel,
        mesh=mesh,
        out_type=[
            jax.ShapeDtypeStruct((NPIX, ROW_W), jnp.float32),
            jax.ShapeDtypeStruct((NPIX, ROW_W), jnp.float32),
        ],
        scratch_types=[
            pltpu.VMEM((BATCH, ROW_W), jnp.float32),
            pltpu.VMEM((BATCH,), jnp.int32),
            pltpu.VMEM((BATCH, ROW_W), jnp.float32),
            pltpu.VMEM((BATCH,), jnp.int32),
            pltpu.VMEM((ZROWS, ROW_W), jnp.float32),
            pltpu.VMEM_SHARED((NPIX, ROW_W), jnp.float32),
            pltpu.SemaphoreType.DMA,
            pltpu.SemaphoreType.DMA,
        ],
        compiler_params=pltpu.CompilerParams(use_tc_tiling_on_sc=False),
    )
    def scatter_kernel(vals_hbm, rows_hbm, ta_hbm, tb_hbm,
                       vbuf0, ibuf0, vbuf1, ibuf1, zbuf, table, sem0, sem1):
        c = lax.axis_index("c")
        s = lax.axis_index("s")
        wid = c * NSUB + s
        sl = NPIX // NSUB

        def zero_row(i, _):
            zbuf[i, :] = jnp.zeros((ROW_W,), jnp.float32)
            return 0

        lax.fori_loop(0, ZROWS, zero_row, 0)
        for j in range(sl // ZROWS):
            pltpu.sync_copy(zbuf, table.at[pl.ds(s * sl + j * ZROWS, ZROWS)])
        plsc.subcore_barrier()

        base = wid * nchunks_per_worker
        slots = ((vbuf0, ibuf0, sem0), (vbuf1, ibuf1, sem1))

        def issue(slot, b):
            vb, ib, sem = slot
            hv = pltpu.async_copy(vals_hbm.at[base + b], vb, sem)
            hi = pltpu.async_copy(rows_hbm.at[base + b], ib, sem)
            return hv, hi

        pending = issue(slots[0], 0)
        for b in range(nchunks_per_worker):
            nxt = None
            if b + 1 < nchunks_per_worker:
                nxt = issue(slots[(b + 1) % 2], b + 1)
            hv, hi = pending
            hv.wait()
            hi.wait()
            vb, ib, _ = slots[b % 2]
            pltpu.sync_copy(vb, table.at[ib], add=True)
            pending = nxt
        plsc.subcore_barrier()

        @pl.when(c == 0)
        def _():
            pltpu.sync_copy(table.at[pl.ds(s * sl, sl)],
                            ta_hbm.at[pl.ds(s * sl, sl)])

        @pl.when(c == 1)
        def _():
            pltpu.sync_copy(table.at[pl.ds(s * sl, sl)],
                            tb_hbm.at[pl.ds(s * sl, sl)])

    return scatter_kernel


def _make_gather(upad, nchunks_per_worker):
    mesh = plsc.VectorSubcoreMesh(core_axis_name="c", subcore_axis_name="s")

    @functools.partial(
        pl.kernel,
        mesh=mesh,
        out_type=jax.ShapeDtypeStruct((upad, ROW_W), jnp.float32),
        scratch_types=[
            pltpu.VMEM((BATCH,), jnp.int32),
            pltpu.VMEM((BATCH, ROW_W), jnp.float32),
            pltpu.VMEM((BATCH, ROW_W), jnp.float32),
            pltpu.VMEM((BATCH, ROW_W), jnp.float32),
            pltpu.SemaphoreType.DMA,
        ],
        compiler_params=pltpu.CompilerParams(use_tc_tiling_on_sc=False),
    )
    def gather_kernel(ta_hbm, tb_hbm, uid_hbm, out_hbm,
                      ibuf, abuf, bbuf, obuf, sem):
        c = lax.axis_index("c")
        s = lax.axis_index("s")
        wid = c * NSUB + s
        for j in range(nchunks_per_worker):
            row0 = (j * NW + wid) * BATCH
            pltpu.sync_copy(uid_hbm.at[pl.ds(row0, BATCH)], ibuf)
            ha = pltpu.async_copy(ta_hbm.at[ibuf], abuf, sem)
            hb = pltpu.async_copy(tb_hbm.at[ibuf], bbuf, sem)
            ha.wait()
            hb.wait()

            def finalize_row(r, _):
                v = (abuf[r, :] + bbuf[r, :]) * 4.0 + 74.0
                obuf[r, :] = jnp.clip(v, 0.0, 255.0)
                return 0

            lax.fori_loop(0, BATCH, finalize_row, 0)
            pltpu.sync_copy(obuf, out_hbm.at[pl.ds(row0, BATCH)])

    return gather_kernel


def kernel(tracks, event_id_map, unique_ids, unique_eventIDs, fields):
    n = tracks.shape[0]
    npad = ((n + 2047) // 2048) * 2048
    u = unique_ids.shape[0]
    upad = ((u + NW * BATCH - 1) // (NW * BATCH)) * (NW * BATCH)

    data = jnp.zeros((npad, 16), jnp.float32)
    data = data.at[:n, :5].set(jnp.take(tracks, fields[:5], axis=1))
    data = data.at[:n, 5].set(event_id_map.astype(jnp.float32))

    bt = 2048
    vals, rows = pl.pallas_call(
        _track_stage,
        grid=(npad // bt,),
        in_specs=[pl.BlockSpec((bt, 16), lambda i: (i, 0))],
        out_specs=[
            pl.BlockSpec((bt, 16 * ROW_W), lambda i: (i, 0)),
            pl.BlockSpec((bt, 16), lambda i: (i, 0)),
        ],
        out_shape=[
            jax.ShapeDtypeStruct((npad, 16 * ROW_W), jnp.float32),
            jax.ShapeDtypeStruct((npad, 16), jnp.int32),
        ],
    )(data)

    ncontrib = npad * 16
    nchunks = ncontrib // BATCH
    vals_r = vals.reshape(nchunks, BATCH, ROW_W)
    rows_r = rows.reshape(nchunks, BATCH)

    ta, tb = _make_scatter(nchunks // NW)(vals_r, rows_r)

    uid_pad = jnp.zeros((upad,), jnp.int32).at[:u].set(unique_ids)
    out16 = _make_gather(upad, upad // (NW * BATCH))(ta, tb, uid_pad)
    return out16[:u, :N_ADC]
